# baseline (speedup 1.0000x reference)
import functools

import jax
import jax.numpy as jnp
from jax import lax
from jax.experimental import pallas as pl
from jax.experimental.pallas import tpu as pltpu
from jax.experimental.pallas import tpu_sc as plsc

BATCH = 16384
EMBED = 64
HIDDEN = 256

NUM_SC = 2
B_PER_SC = BATCH // NUM_SC
IDX_CHUNK = 2048


def _make_gather_kernel():
    mesh = plsc.ScalarSubcoreMesh(axis_name="core", num_cores=NUM_SC)
    out_type = (
        jax.ShapeDtypeStruct((BATCH, EMBED), jnp.float32),
        jax.ShapeDtypeStruct((BATCH, EMBED), jnp.float32),
    )

    @functools.partial(
        pl.kernel,
        mesh=mesh,
        out_type=out_type,
        scratch_types=[
            pltpu.SMEM((IDX_CHUNK,), jnp.int32),
            pltpu.SMEM((IDX_CHUNK,), jnp.int32),
            pltpu.SemaphoreType.DMA,
            pltpu.SemaphoreType.DMA,
            pltpu.SemaphoreType.DMA,
        ],
    )
    def gather_kernel(
        user_table_hbm,
        movie_table_hbm,
        users_hbm,
        movies_hbm,
        out_u_hbm,
        out_m_hbm,
        idx_u,
        idx_m,
        sem_i,
        sem_u,
        sem_m,
    ):
        cid = lax.axis_index("core")
        base = cid * B_PER_SC
        for c in range(B_PER_SC // IDX_CHUNK):
            base_c = base + c * IDX_CHUNK
            pltpu.async_copy(
                users_hbm.at[pl.ds(base_c, IDX_CHUNK)], idx_u, sem_i
            ).wait()
            pltpu.async_copy(
                movies_hbm.at[pl.ds(base_c, IDX_CHUNK)], idx_m, sem_i
            ).wait()

            @pl.loop(0, IDX_CHUNK)
            def _(i):
                iu = idx_u[i]
                im = idx_m[i]
                pltpu.async_copy(
                    user_table_hbm.at[pl.ds(iu, 1)],
                    out_u_hbm.at[pl.ds(base_c + i, 1)],
                    sem_u,
                )
                pltpu.async_copy(
                    movie_table_hbm.at[pl.ds(im, 1)],
                    out_m_hbm.at[pl.ds(base_c + i, 1)],
                    sem_m,
                )

        pltpu.make_async_copy(
            user_table_hbm.at[pl.ds(0, B_PER_SC)],
            out_u_hbm.at[pl.ds(base, B_PER_SC)],
            sem_u,
        ).wait()
        pltpu.make_async_copy(
            movie_table_hbm.at[pl.ds(0, B_PER_SC)],
            out_m_hbm.at[pl.ds(base, B_PER_SC)],
            sem_m,
        ).wait()

    return gather_kernel


@functools.lru_cache(maxsize=1)
def _get_gather():
    return _make_gather_kernel()


def _mlp_body(u_ref, m_ref, w1u_ref, w1m_ref, b1_ref, w2_ref, b2_ref, o_ref):
    h = (
        jnp.dot(u_ref[...], w1u_ref[...], preferred_element_type=jnp.float32)
        + jnp.dot(m_ref[...], w1m_ref[...], preferred_element_type=jnp.float32)
        + b1_ref[...]
    )
    h = jnp.maximum(h, 0.0)
    o_ref[...] = jnp.sum(h * w2_ref[...], axis=1, keepdims=True) + b2_ref[...]


def _mlp(u, m, w1u, w1m, b1_2d, w2, b2_2d, block_rows=2048):
    grid = (BATCH // block_rows,)
    return pl.pallas_call(
        _mlp_body,
        grid=grid,
        in_specs=[
            pl.BlockSpec((block_rows, EMBED), lambda i: (i, 0)),
            pl.BlockSpec((block_rows, EMBED), lambda i: (i, 0)),
            pl.BlockSpec((EMBED, HIDDEN), lambda i: (0, 0)),
            pl.BlockSpec((EMBED, HIDDEN), lambda i: (0, 0)),
            pl.BlockSpec((1, HIDDEN), lambda i: (0, 0)),
            pl.BlockSpec((1, HIDDEN), lambda i: (0, 0)),
            pl.BlockSpec((1, 1), lambda i: (0, 0)),
        ],
        out_specs=pl.BlockSpec((block_rows, 1), lambda i: (i, 0)),
        out_shape=jax.ShapeDtypeStruct((BATCH, 1), jnp.float32),
    )(u, m, w1u, w1m, b1_2d, w2, b2_2d)


@jax.jit
def kernel(users, movies, user_table, movie_table, W1, b1, W2, b2):
    users = users.astype(jnp.int32)
    movies = movies.astype(jnp.int32)
    u_rows, m_rows = _get_gather()(user_table, movie_table, users, movies)
    w1t = W1.T
    w1u = w1t[:EMBED]
    w1m = w1t[EMBED:]
    b1_2d = b1.reshape(1, HIDDEN)
    b2_2d = b2.reshape(1, 1)
    return _mlp(u_rows, m_rows, w1u, w1m, b1_2d, W2, b2_2d)
